# Initial kernel scaffold; baseline (speedup 1.0000x reference)
import jax
import jax.numpy as jnp
from jax.experimental import pallas as pl


def kernel(in_feat, edge_index, W1, b1, W2, b2):
    raise NotImplementedError("write your pallas kernel here")


# speedup vs baseline: 4.8456x; 4.8456x over previous
import functools

import jax
import jax.numpy as jnp
from jax import lax
from jax.experimental import pallas as pl
from jax.experimental.pallas import tpu as pltpu
from jax.experimental.pallas import tpu_sc as plsc

N_NODES = 10000
N_PAD = 10240
N_EDGES = 320000
TILES = 16
CHUNKS = 250
K = 80
CHUNKS_OUT = 10
CHUNKS_IN = CHUNKS // CHUNKS_OUT
ROWS_PER_TILE = N_PAD // TILES

_MESH = plsc.VectorSubcoreMesh(core_axis_name="c", subcore_axis_name="s")
_SC_PARAMS = pltpu.CompilerParams(use_tc_tiling_on_sc=False)


def _degrees(src3, dst3, ones8, zeros8):
    @functools.partial(
        pl.kernel,
        out_type=(
            jax.ShapeDtypeStruct((N_PAD, 8), jnp.float32),
            jax.ShapeDtypeStruct((N_PAD, 8), jnp.float32),
        ),
        mesh=_MESH,
        compiler_params=_SC_PARAMS,
        scratch_types=[
            pltpu.VMEM_SHARED((N_PAD, 8), jnp.float32),
            pltpu.VMEM((CHUNKS, K), jnp.int32),
            pltpu.VMEM((K, 8), jnp.float32),
        ],
    )
    def deg_kernel(src_hbm, dst_hbm, ones_hbm, zeros_hbm,
                   outdeg_hbm, indeg_hbm, hist, idx_v, ones_v):
        c = lax.axis_index("c")
        s = lax.axis_index("s")

        @pl.when(c == 0)
        def _():
            pltpu.sync_copy(src_hbm.at[s], idx_v)

        @pl.when(c == 1)
        def _():
            pltpu.sync_copy(dst_hbm.at[s], idx_v)

        pltpu.sync_copy(ones_hbm, ones_v)
        rows = pl.ds(s * ROWS_PER_TILE, ROWS_PER_TILE)
        pltpu.sync_copy(zeros_hbm.at[rows], hist.at[rows])
        plsc.subcore_barrier()

        def scatter(g, carry):
            pltpu.sync_copy(ones_v, hist.at[idx_v.at[g]], add=True)
            return carry

        lax.fori_loop(0, CHUNKS, scatter, 0)
        plsc.subcore_barrier()

        @pl.when(c == 0)
        def _():
            pltpu.sync_copy(hist.at[rows], outdeg_hbm.at[rows])

        @pl.when(c == 1)
        def _():
            pltpu.sync_copy(hist.at[rows], indeg_hbm.at[rows])

    return deg_kernel(src3, dst3, ones8, zeros8)


def _aggregate(xw0, xw1, src3, dst3, zerosD, D):
    @functools.partial(
        pl.kernel,
        out_type=(
            jax.ShapeDtypeStruct((N_PAD, D), jnp.float32),
            jax.ShapeDtypeStruct((N_PAD, D), jnp.float32),
        ),
        mesh=_MESH,
        compiler_params=_SC_PARAMS,
        scratch_types=[
            pltpu.VMEM_SHARED((N_PAD, D), jnp.float32),
            pltpu.VMEM((CHUNKS_IN, K), jnp.int32),
            pltpu.VMEM((CHUNKS_IN, K), jnp.int32),
            pltpu.VMEM((K, D), jnp.float32),
            pltpu.SemaphoreType.DMA,
        ],
    )
    def agg_kernel(xw0_hbm, xw1_hbm, s_hbm, d_hbm, z_hbm, out0_hbm, out1_hbm,
                   acc, src_v, dst_v, rows_v, sem):
        c = lax.axis_index("c")
        s = lax.axis_index("s")

        zrows_slice = pl.ds(s * ROWS_PER_TILE, ROWS_PER_TILE)
        pltpu.sync_copy(z_hbm.at[zrows_slice], acc.at[zrows_slice])
        plsc.subcore_barrier()

        def make_loop(xw_hbm):
            def outer(o, carry):
                pltpu.sync_copy(s_hbm.at[s, o], src_v)
                pltpu.sync_copy(d_hbm.at[s, o], dst_v)

                def body(g, carry2):
                    pltpu.async_copy(xw_hbm.at[src_v.at[g]], rows_v, sem).wait()
                    pltpu.sync_copy(rows_v, acc.at[dst_v.at[g]], add=True)
                    return carry2

                lax.fori_loop(0, CHUNKS_IN, body, 0)
                return carry
            return outer

        @pl.when(c == 0)
        def _():
            lax.fori_loop(0, CHUNKS_OUT, make_loop(xw0_hbm), 0)

        @pl.when(c == 1)
        def _():
            lax.fori_loop(0, CHUNKS_OUT, make_loop(xw1_hbm), 0)

        plsc.subcore_barrier()
        rows = pl.ds(s * ROWS_PER_TILE, ROWS_PER_TILE)

        @pl.when(c == 0)
        def _():
            pltpu.sync_copy(acc.at[rows], out0_hbm.at[rows])

        @pl.when(c == 1)
        def _():
            pltpu.sync_copy(acc.at[rows], out1_hbm.at[rows])

    return agg_kernel(xw0, xw1, src3, dst3, zerosD)


def _norm_col(deg16):
    return lax.rsqrt(jnp.maximum(deg16[:, 0:1], 1.0))


def _mm1(x, W1, degO16):
    M, KIN, H = N_NODES, x.shape[1], W1.shape[1]
    BM = 1000

    def body(x_ref, w_ref, dego_ref, xw0_ref, xw1_ref):
        on = _norm_col(dego_ref[...])
        xs = x_ref[...] * on
        xw = lax.dot_general(xs, w_ref[...], (((1,), (0,)), ((), ())),
                             precision=lax.Precision.HIGHEST,
                             preferred_element_type=jnp.float32)
        xw0_ref[...] = xw[:, : H // 2]
        xw1_ref[...] = xw[:, H // 2:]

    return pl.pallas_call(
        body,
        grid=(M // BM,),
        in_specs=[
            pl.BlockSpec((BM, KIN), lambda i: (i, 0)),
            pl.BlockSpec((KIN, H), lambda i: (0, 0)),
            pl.BlockSpec((BM, 8), lambda i: (i, 0)),
        ],
        out_specs=[
            pl.BlockSpec((BM, H // 2), lambda i: (i, 0)),
            pl.BlockSpec((BM, H // 2), lambda i: (i, 0)),
        ],
        out_shape=[
            jax.ShapeDtypeStruct((M, H // 2), jnp.float32),
            jax.ShapeDtypeStruct((M, H // 2), jnp.float32),
        ],
    )(x, W1, degO16)


def _mm2(a0, a1, degI16, degO16, b1row, W2):
    M, H = N_NODES, 2 * a0.shape[1]
    OUT = W2.shape[1]
    BM = 1000

    def body(a0_ref, a1_ref, degi_ref, dego_ref, b_ref, w_ref, o0_ref, o1_ref):
        inn = _norm_col(degi_ref[...])
        on = _norm_col(dego_ref[...])
        h0 = jnp.maximum(a0_ref[...] * inn + b_ref[0:1, : H // 2], 0.0) * on
        h1 = jnp.maximum(a1_ref[...] * inn + b_ref[0:1, H // 2:], 0.0) * on
        dn = (((1,), (0,)), ((), ()))
        xw = lax.dot_general(h0, w_ref[: H // 2, :], dn,
                             precision=lax.Precision.HIGHEST,
                             preferred_element_type=jnp.float32)
        xw = xw + lax.dot_general(h1, w_ref[H // 2:, :], dn,
                                  precision=lax.Precision.HIGHEST,
                                  preferred_element_type=jnp.float32)
        o0_ref[...] = xw[:, : OUT // 2]
        o1_ref[...] = xw[:, OUT // 2:]

    return pl.pallas_call(
        body,
        grid=(M // BM,),
        in_specs=[
            pl.BlockSpec((BM, H // 2), lambda i: (i, 0)),
            pl.BlockSpec((BM, H // 2), lambda i: (i, 0)),
            pl.BlockSpec((BM, 8), lambda i: (i, 0)),
            pl.BlockSpec((BM, 8), lambda i: (i, 0)),
            pl.BlockSpec((1, H), lambda i: (0, 0)),
            pl.BlockSpec((H, OUT), lambda i: (0, 0)),
        ],
        out_specs=[
            pl.BlockSpec((BM, OUT // 2), lambda i: (i, 0)),
            pl.BlockSpec((BM, OUT // 2), lambda i: (i, 0)),
        ],
        out_shape=[
            jax.ShapeDtypeStruct((M, OUT // 2), jnp.float32),
            jax.ShapeDtypeStruct((M, OUT // 2), jnp.float32),
        ],
    )(a0, a1, degI16, degO16, b1row, W2)


def _final(a0, a1, degI16, b2row):
    M, OUT = N_NODES, 2 * a0.shape[1]
    BM = 1000

    def body(a0_ref, a1_ref, degi_ref, b_ref, out_ref):
        i = pl.program_id(0)
        inn = _norm_col(degi_ref[...])
        h0 = jnp.maximum(a0_ref[...] * inn + b_ref[0:1, : OUT // 2], 0.0)
        h1 = jnp.maximum(a1_ref[...] * inn + b_ref[0:1, OUT // 2:], 0.0)
        part = jnp.concatenate(
            [jnp.sum(h0, axis=0, keepdims=True),
             jnp.sum(h1, axis=0, keepdims=True)], axis=1) * (1.0 / M)

        @pl.when(i == 0)
        def _():
            out_ref[...] = jnp.zeros_like(out_ref)

        out_ref[...] += part

    return pl.pallas_call(
        body,
        grid=(M // BM,),
        in_specs=[
            pl.BlockSpec((BM, OUT // 2), lambda i: (i, 0)),
            pl.BlockSpec((BM, OUT // 2), lambda i: (i, 0)),
            pl.BlockSpec((BM, 8), lambda i: (i, 0)),
            pl.BlockSpec((1, OUT), lambda i: (0, 0)),
        ],
        out_specs=pl.BlockSpec((1, OUT), lambda i: (0, 0)),
        out_shape=jax.ShapeDtypeStruct((1, OUT), jnp.float32),
    )(a0, a1, degI16, b2row)


def kernel(in_feat, edge_index, W1, b1, W2, b2):
    src3 = edge_index[0].astype(jnp.int32).reshape(TILES, CHUNKS, K)
    dst3 = edge_index[1].astype(jnp.int32).reshape(TILES, CHUNKS, K)
    src4 = src3.reshape(TILES, CHUNKS_OUT, CHUNKS_IN, K)
    dst4 = dst3.reshape(TILES, CHUNKS_OUT, CHUNKS_IN, K)
    ones8 = jnp.ones((K, 8), jnp.float32)
    zeros8 = jnp.zeros((N_PAD, 8), jnp.float32)
    degO16p, degI16p = _degrees(src3, dst3, ones8, zeros8)
    degO16, degI16 = degO16p[:N_NODES], degI16p[:N_NODES]
    xw0, xw1 = _mm1(in_feat, W1, degO16)
    zh = jnp.zeros((N_PAD, W1.shape[1] // 2), jnp.float32)
    a10, a11 = _aggregate(xw0, xw1, src4, dst4, zh, W1.shape[1] // 2)
    xv0, xv1 = _mm2(a10[:N_NODES], a11[:N_NODES], degI16, degO16,
                    b1.reshape(1, -1), W2)
    zo = jnp.zeros((N_PAD, W2.shape[1] // 2), jnp.float32)
    a20, a21 = _aggregate(xv0, xv1, src4, dst4, zo, W2.shape[1] // 2)
    out = _final(a20[:N_NODES], a21[:N_NODES], degI16, b2.reshape(1, -1))
    return out.reshape(W2.shape[1])


# speedup vs baseline: 6.6257x; 1.3674x over previous
import functools

import jax
import jax.numpy as jnp
from jax import lax
from jax.experimental import pallas as pl
from jax.experimental.pallas import tpu as pltpu
from jax.experimental.pallas import tpu_sc as plsc

N_NODES = 10000
N_PAD = 10240
N_EDGES = 320000
TILES = 16
CHUNKS = 250
K = 80
AG_K = 100
AG_CI = 40
AG_CO = 5
ROWS_PER_TILE = N_PAD // TILES

_MESH = plsc.VectorSubcoreMesh(core_axis_name="c", subcore_axis_name="s")
_SC_PARAMS = pltpu.CompilerParams(use_tc_tiling_on_sc=False)


def _degrees(src3, dst3, ones8, zeros8):
    @functools.partial(
        pl.kernel,
        out_type=(
            jax.ShapeDtypeStruct((N_PAD, 8), jnp.float32),
            jax.ShapeDtypeStruct((N_PAD, 8), jnp.float32),
        ),
        mesh=_MESH,
        compiler_params=_SC_PARAMS,
        scratch_types=[
            pltpu.VMEM_SHARED((N_PAD, 8), jnp.float32),
            pltpu.VMEM((CHUNKS, K), jnp.int32),
            pltpu.VMEM((K, 8), jnp.float32),
        ],
    )
    def deg_kernel(src_hbm, dst_hbm, ones_hbm, zeros_hbm,
                   outdeg_hbm, indeg_hbm, hist, idx_v, ones_v):
        c = lax.axis_index("c")
        s = lax.axis_index("s")

        @pl.when(c == 0)
        def _():
            pltpu.sync_copy(src_hbm.at[s], idx_v)

        @pl.when(c == 1)
        def _():
            pltpu.sync_copy(dst_hbm.at[s], idx_v)

        pltpu.sync_copy(ones_hbm, ones_v)
        rows = pl.ds(s * ROWS_PER_TILE, ROWS_PER_TILE)
        pltpu.sync_copy(zeros_hbm.at[rows], hist.at[rows])
        plsc.subcore_barrier()

        def scatter(g, carry):
            pltpu.sync_copy(ones_v, hist.at[idx_v.at[g]], add=True)
            return carry

        lax.fori_loop(0, CHUNKS, scatter, 0)
        plsc.subcore_barrier()

        @pl.when(c == 0)
        def _():
            pltpu.sync_copy(hist.at[rows], outdeg_hbm.at[rows])

        @pl.when(c == 1)
        def _():
            pltpu.sync_copy(hist.at[rows], indeg_hbm.at[rows])

    return deg_kernel(src3, dst3, ones8, zeros8)


def _aggregate(xw0, xw1, src4, dst4, zerosD, D):
    NT = AG_CI // 2

    @functools.partial(
        pl.kernel,
        out_type=(
            jax.ShapeDtypeStruct((N_PAD, D), jnp.float32),
            jax.ShapeDtypeStruct((N_PAD, D), jnp.float32),
        ),
        mesh=_MESH,
        compiler_params=_SC_PARAMS,
        scratch_types=[
            pltpu.VMEM_SHARED((N_PAD, D), jnp.float32),
            pltpu.VMEM((AG_CI, AG_K), jnp.int32),
            pltpu.VMEM((AG_CI, AG_K), jnp.int32),
            pltpu.VMEM((AG_K, D), jnp.float32),
            pltpu.VMEM((AG_K, D), jnp.float32),
            pltpu.SemaphoreType.DMA,
            pltpu.SemaphoreType.DMA,
            pltpu.SemaphoreType.DMA,
            pltpu.SemaphoreType.DMA,
        ],
    )
    def agg_kernel(xw0_hbm, xw1_hbm, s_hbm, d_hbm, z_hbm, out0_hbm, out1_hbm,
                   acc, src_v, dst_v, rows0, rows1, g0, g1, s0, s1):
        c = lax.axis_index("c")
        s = lax.axis_index("s")

        zrows_slice = pl.ds(s * ROWS_PER_TILE, ROWS_PER_TILE)
        pltpu.sync_copy(z_hbm.at[zrows_slice], acc.at[zrows_slice])
        plsc.subcore_barrier()

        def run_edges(xw_hbm):
            def gather(chunk, buf, sem):
                pltpu.async_copy(xw_hbm.at[src_v.at[chunk]], buf, sem)

            def gwait(buf, sem):
                pltpu.make_async_copy(xw_hbm.at[src_v.at[0]], buf, sem).wait()

            def scat(chunk, buf, sem):
                pltpu.async_copy(buf, acc.at[dst_v.at[chunk]], sem, add=True)

            def swait(buf, sem):
                pltpu.make_async_copy(buf, acc.at[dst_v.at[0]], sem).wait()

            def outer(o, carry):
                pltpu.sync_copy(s_hbm.at[s, o], src_v)
                pltpu.sync_copy(d_hbm.at[s, o], dst_v)
                gather(0, rows0, g0)

                def body(t, carry2):
                    a = 2 * t
                    gwait(rows0, g0)

                    @pl.when(t > 0)
                    def _():
                        swait(rows1, s1)

                    gather(a + 1, rows1, g1)
                    scat(a, rows0, s0)
                    gwait(rows1, g1)
                    swait(rows0, s0)

                    @pl.when(t < NT - 1)
                    def _():
                        gather(a + 2, rows0, g0)

                    scat(a + 1, rows1, s1)
                    return carry2

                lax.fori_loop(0, NT, body, 0)
                swait(rows1, s1)
                return carry

            lax.fori_loop(0, AG_CO, outer, 0)

        @pl.when(c == 0)
        def _():
            run_edges(xw0_hbm)

        @pl.when(c == 1)
        def _():
            run_edges(xw1_hbm)

        plsc.subcore_barrier()
        rows = pl.ds(s * ROWS_PER_TILE, ROWS_PER_TILE)

        @pl.when(c == 0)
        def _():
            pltpu.sync_copy(acc.at[rows], out0_hbm.at[rows])

        @pl.when(c == 1)
        def _():
            pltpu.sync_copy(acc.at[rows], out1_hbm.at[rows])

    return agg_kernel(xw0, xw1, src4, dst4, zerosD)


def _norm_col(deg16):
    return lax.rsqrt(jnp.maximum(deg16[:, 0:1], 1.0))


def _mm1(x, W1, degO16):
    M, KIN, H = N_NODES, x.shape[1], W1.shape[1]
    BM = 1000

    def body(x_ref, w_ref, dego_ref, xw0_ref, xw1_ref):
        on = _norm_col(dego_ref[...])
        xs = x_ref[...] * on
        xw = lax.dot_general(xs, w_ref[...], (((1,), (0,)), ((), ())),
                             precision=lax.Precision.HIGHEST,
                             preferred_element_type=jnp.float32)
        xw0_ref[...] = xw[:, : H // 2]
        xw1_ref[...] = xw[:, H // 2:]

    return pl.pallas_call(
        body,
        grid=(M // BM,),
        in_specs=[
            pl.BlockSpec((BM, KIN), lambda i: (i, 0)),
            pl.BlockSpec((KIN, H), lambda i: (0, 0)),
            pl.BlockSpec((BM, 8), lambda i: (i, 0)),
        ],
        out_specs=[
            pl.BlockSpec((BM, H // 2), lambda i: (i, 0)),
            pl.BlockSpec((BM, H // 2), lambda i: (i, 0)),
        ],
        out_shape=[
            jax.ShapeDtypeStruct((M, H // 2), jnp.float32),
            jax.ShapeDtypeStruct((M, H // 2), jnp.float32),
        ],
    )(x, W1, degO16)


def _mm2(a0, a1, degI16, degO16, b1row, W2):
    M, H = N_NODES, 2 * a0.shape[1]
    OUT = W2.shape[1]
    BM = 1000

    def body(a0_ref, a1_ref, degi_ref, dego_ref, b_ref, w_ref, o0_ref, o1_ref):
        inn = _norm_col(degi_ref[...])
        on = _norm_col(dego_ref[...])
        h0 = jnp.maximum(a0_ref[...] * inn + b_ref[0:1, : H // 2], 0.0) * on
        h1 = jnp.maximum(a1_ref[...] * inn + b_ref[0:1, H // 2:], 0.0) * on
        dn = (((1,), (0,)), ((), ()))
        xw = lax.dot_general(h0, w_ref[: H // 2, :], dn,
                             precision=lax.Precision.HIGHEST,
                             preferred_element_type=jnp.float32)
        xw = xw + lax.dot_general(h1, w_ref[H // 2:, :], dn,
                                  precision=lax.Precision.HIGHEST,
                                  preferred_element_type=jnp.float32)
        o0_ref[...] = xw[:, : OUT // 2]
        o1_ref[...] = xw[:, OUT // 2:]

    return pl.pallas_call(
        body,
        grid=(M // BM,),
        in_specs=[
            pl.BlockSpec((BM, H // 2), lambda i: (i, 0)),
            pl.BlockSpec((BM, H // 2), lambda i: (i, 0)),
            pl.BlockSpec((BM, 8), lambda i: (i, 0)),
            pl.BlockSpec((BM, 8), lambda i: (i, 0)),
            pl.BlockSpec((1, H), lambda i: (0, 0)),
            pl.BlockSpec((H, OUT), lambda i: (0, 0)),
        ],
        out_specs=[
            pl.BlockSpec((BM, OUT // 2), lambda i: (i, 0)),
            pl.BlockSpec((BM, OUT // 2), lambda i: (i, 0)),
        ],
        out_shape=[
            jax.ShapeDtypeStruct((M, OUT // 2), jnp.float32),
            jax.ShapeDtypeStruct((M, OUT // 2), jnp.float32),
        ],
    )(a0, a1, degI16, degO16, b1row, W2)


def _final(a0, a1, degI16, b2row):
    M, OUT = N_NODES, 2 * a0.shape[1]
    BM = 1000

    def body(a0_ref, a1_ref, degi_ref, b_ref, out_ref):
        i = pl.program_id(0)
        inn = _norm_col(degi_ref[...])
        h0 = jnp.maximum(a0_ref[...] * inn + b_ref[0:1, : OUT // 2], 0.0)
        h1 = jnp.maximum(a1_ref[...] * inn + b_ref[0:1, OUT // 2:], 0.0)
        part = jnp.concatenate(
            [jnp.sum(h0, axis=0, keepdims=True),
             jnp.sum(h1, axis=0, keepdims=True)], axis=1) * (1.0 / M)

        @pl.when(i == 0)
        def _():
            out_ref[...] = jnp.zeros_like(out_ref)

        out_ref[...] += part

    return pl.pallas_call(
        body,
        grid=(M // BM,),
        in_specs=[
            pl.BlockSpec((BM, OUT // 2), lambda i: (i, 0)),
            pl.BlockSpec((BM, OUT // 2), lambda i: (i, 0)),
            pl.BlockSpec((BM, 8), lambda i: (i, 0)),
            pl.BlockSpec((1, OUT), lambda i: (0, 0)),
        ],
        out_specs=pl.BlockSpec((1, OUT), lambda i: (0, 0)),
        out_shape=jax.ShapeDtypeStruct((1, OUT), jnp.float32),
    )(a0, a1, degI16, b2row)


def kernel(in_feat, edge_index, W1, b1, W2, b2):
    src3 = edge_index[0].astype(jnp.int32).reshape(TILES, CHUNKS, K)
    dst3 = edge_index[1].astype(jnp.int32).reshape(TILES, CHUNKS, K)
    src4 = edge_index[0].astype(jnp.int32).reshape(TILES, AG_CO, AG_CI, AG_K)
    dst4 = edge_index[1].astype(jnp.int32).reshape(TILES, AG_CO, AG_CI, AG_K)
    ones8 = jnp.ones((K, 8), jnp.float32)
    zeros8 = jnp.zeros((N_PAD, 8), jnp.float32)
    degO16p, degI16p = _degrees(src3, dst3, ones8, zeros8)
    degO16, degI16 = degO16p[:N_NODES], degI16p[:N_NODES]
    xw0, xw1 = _mm1(in_feat, W1, degO16)
    zh = jnp.zeros((N_PAD, W1.shape[1] // 2), jnp.float32)
    a10, a11 = _aggregate(xw0, xw1, src4, dst4, zh, W1.shape[1] // 2)
    xv0, xv1 = _mm2(a10[:N_NODES], a11[:N_NODES], degI16, degO16,
                    b1.reshape(1, -1), W2)
    zo = jnp.zeros((N_PAD, W2.shape[1] // 2), jnp.float32)
    a20, a21 = _aggregate(xv0, xv1, src4, dst4, zo, W2.shape[1] // 2)
    out = _final(a20[:N_NODES], a21[:N_NODES], degI16, b2.reshape(1, -1))
    return out.reshape(W2.shape[1])
